# baseline (speedup 1.0000x reference)
import functools

import jax
import jax.numpy as jnp
from jax import lax
from jax.experimental import pallas as pl
from jax.experimental.pallas import tpu as pltpu
from jax.experimental.pallas import tpu_sc as plsc

_K = 64
_N = 4096
_GROUPS = 256
_SLABS = _N // _GROUPS


def _f32_to_key(x):
    bits = lax.bitcast_convert_type(x, jnp.int32)
    return jnp.where(bits < 0, bits ^ jnp.int32(0x7FFFFFFF), bits)


def _key_to_f32(k):
    return lax.bitcast_convert_type(
        jnp.where(k < 0, k ^ jnp.int32(0x7FFFFFFF), k), jnp.float32)


def _ceil_mid(lo, hi):
    return (lo >> 1) + (hi >> 1) + ((lo | hi) & 1)


def _prescreen_body(x_ref, o_ref):
    x = x_ref[...]
    r = x.shape[0]
    acc = x[:, 0:_GROUPS]
    for k in range(1, _SLABS):
        acc = jnp.maximum(acc, x[:, k * _GROUPS:(k + 1) * _GROUPS])
    key = _f32_to_key(acc)

    def step(_, lohi):
        lo, hi = lohi
        mid = _ceil_mid(lo, hi)
        cnt = jnp.sum((key >= mid).astype(jnp.float32), axis=1, keepdims=True)
        ok = cnt >= jnp.float32(_K)
        return (jnp.where(ok, mid, lo), jnp.where(ok, hi, mid - jnp.int32(1)))

    lo0 = jnp.full((r, 1), -0x80000000, jnp.int32)
    hi0 = jnp.full((r, 1), 0x7FFFFFFF, jnp.int32)
    lo, _ = jax.lax.fori_loop(0, 16, step, (lo0, hi0))
    o_ref[...] = jnp.broadcast_to(lo, (r, 16))


def _prescreen(x2):
    rows = x2.shape[0]
    blk = 256
    return pl.pallas_call(
        _prescreen_body,
        grid=(rows // blk,),
        in_specs=[pl.BlockSpec((blk, _N), lambda i: (i, 0))],
        out_specs=pl.BlockSpec((blk, 16), lambda i: (i, 0)),
        out_shape=jax.ShapeDtypeStruct((rows, 16), jnp.int32),
    )(x2)


def _sc_topk(x2d, thr_flat, rows):
    nw = 32
    rows_per = rows // nw
    pairs = rows_per // 2
    mesh = plsc.VectorSubcoreMesh(core_axis_name="c", subcore_axis_name="s")

    @functools.partial(
        pl.kernel,
        mesh=mesh,
        compiler_params=pltpu.CompilerParams(
            needs_layout_passes=False, use_tc_tiling_on_sc=False),
        out_type=jax.ShapeDtypeStruct((rows, _K), jnp.float32),
        scratch_types=[
            pltpu.VMEM((_N,), jnp.float32),
            pltpu.VMEM((_N,), jnp.float32),
            pltpu.VMEM((_N,), jnp.int32),
            pltpu.VMEM((rows_per * 16,), jnp.int32),
            pltpu.VMEM((64, _K), jnp.float32),
            pltpu.VMEM((16,), jnp.int32),
            pltpu.SemaphoreType.DMA,
            pltpu.SemaphoreType.DMA,
        ],
    )
    def sck(x_hbm, thr_hbm, out_hbm, buf0, buf1, cand, thr_v, outstage,
            cnt_v, sem0, sem1):
        wid = lax.axis_index("s") * 2 + lax.axis_index("c")
        row0 = wid * rows_per
        lanes = lax.iota(jnp.int32, 16)
        zero_i = jnp.zeros((16,), jnp.int32)
        one_i = jnp.full((16,), 1, jnp.int32)
        k64 = jnp.full((16,), _K, jnp.int32)
        fifteens = jnp.full((16,), 15, jnp.int32)
        int_min = jnp.full((16,), -0x80000000, jnp.int32)
        int_max = jnp.full((16,), 0x7FFFFFFF, jnp.int32)
        j128 = jnp.full((16,), 128, jnp.int32)

        pltpu.sync_copy(thr_hbm.at[pl.ds(row0 * 16, rows_per * 16)], thr_v)

        def row_src(row):
            return x_hbm.at[row]

        def splat_last(c):
            return lax.gather(
                c, fifteens[:, None],
                dimension_numbers=lax.GatherDimensionNumbers(
                    offset_dims=(), collapsed_slice_dims=(0,),
                    start_index_map=(0,)),
                slice_sizes=(1,),
                mode=lax.GatherScatterMode.PROMISE_IN_BOUNDS)

        def process(buf, row, rm_vec):
            rloc = row - row0
            lk = thr_v[pl.ds(rloc * 16, 16)]
            for q in range(8):
                cand[pl.ds(q * 16, 16)] = int_min

            def hot(jo, carry):
                base, mx = carry
                for u in range(8):
                    x = buf[pl.ds(jo * 128 + u * 16, 16)]
                    uk = _f32_to_key(x)
                    m = uk >= lk
                    mx = jnp.maximum(mx, uk)
                    c = plsc.cumsum(jnp.where(m, jnp.int32(1), jnp.int32(0)))
                    idx = base + c - 1
                    plsc.store_scatter(cand, [idx], uk, mask=m)
                    base = base + splat_last(c)
                return (base, mx)

            nsplat, mx = jax.lax.fori_loop(0, _N // 128, hot,
                                           (zero_i, int_min))
            cmax = splat_last(plsc.cummax(mx))
            nceil = ((nsplat + 15) >> 4) << 4
            padidx = lanes + nceil - 16
            plsc.store_scatter(cand, [padidx], int_min, mask=padidx >= nsplat)

            def count_ge(mid):
                acc = zero_i
                for q in range(8):
                    kv = cand[pl.ds(q * 16, 16)]
                    acc = acc + jnp.where(kv >= mid, jnp.int32(1),
                                          jnp.int32(0))

                def cond(st):
                    return jnp.any(st[0] < nceil)

                def body(st):
                    j, a = st
                    kv = plsc.load_gather(cand, [lanes + j])
                    return (j + 16,
                            a + jnp.where(kv >= mid, jnp.int32(1),
                                          jnp.int32(0)))

                _, acc = jax.lax.while_loop(cond, body, (j128, acc))
                return splat_last(plsc.cumsum(acc))

            def bstep(_, lohi):
                lo, hi = lohi
                mid = _ceil_mid(lo, hi)
                ok = count_ge(mid) >= k64
                return (jnp.where(ok, mid, lo),
                        jnp.where(ok, hi, mid - jnp.int32(1)))

            tkey, _ = jax.lax.while_loop(
                lambda lohi: jnp.any(lohi[0] < lohi[1]),
                lambda lohi: bstep(0, lohi), (lk, cmax))
            budget = k64 - count_ge(tkey + 1)

            def emit(kv, eqb, ob):
                gt = kv > tkey
                eq = kv == tkey
                eqc = plsc.cumsum(jnp.where(eq, jnp.int32(1),
                                            jnp.int32(0))) + eqb
                keep = gt | (eq & (eqc <= budget))
                pc = plsc.cumsum(jnp.where(keep, jnp.int32(1), jnp.int32(0)))
                pos = ob + pc - 1
                plsc.store_scatter(outstage, [rm_vec >> 6, pos],
                                   _key_to_f32(kv), mask=keep)
                return splat_last(eqc), ob + splat_last(pc)

            eqb = zero_i
            ob = zero_i
            for q in range(8):
                eqb, ob = emit(cand[pl.ds(q * 16, 16)], eqb, ob)

            def fcond(st):
                return jnp.any(st[0] < nceil)

            def fbody(st):
                j, eqb, ob = st
                kv = plsc.load_gather(cand, [lanes + j])
                eqb, ob = emit(kv, eqb, ob)
                return (j + 16, eqb, ob)

            jax.lax.while_loop(fcond, fbody, (j128, eqb, ob))

            @pl.when((row & 63) == 63)
            def _():
                pltpu.sync_copy(
                    outstage, out_hbm.at[pl.ds(row - 63, 64)])

        pltpu.async_copy(row_src(row0), buf0, sem0)

        def pair(i, rm_vec):
            row_a = row0 + 2 * i
            pltpu.make_async_copy(row_src(row_a), buf0, sem0).wait()
            pltpu.async_copy(row_src(row_a + 1), buf1, sem1)
            process(buf0, row_a, rm_vec)
            pltpu.make_async_copy(row_src(row_a + 1), buf1, sem1).wait()

            @pl.when(i < pairs - 1)
            def _():
                pltpu.async_copy(row_src(row_a + 2), buf0, sem0)

            process(buf1, row_a + 1, rm_vec + _K)
            return (rm_vec + 2 * _K) & jnp.int32(_N - 1)

        jax.lax.fori_loop(0, pairs, pair, zero_i)

    return sck(x2d, thr_flat)


def kernel(inputs):
    b, s, n = inputs.shape
    rows = b * s
    x2 = inputs.reshape(rows, n)
    thr = _prescreen(x2).reshape(-1)
    out = _sc_topk(x2, thr, rows)
    return out.reshape(b, s, _K)


# speedup vs baseline: 1.0777x; 1.0777x over previous
import functools

import jax
import jax.numpy as jnp
from jax import lax
from jax.experimental import pallas as pl
from jax.experimental.pallas import tpu as pltpu
from jax.experimental.pallas import tpu_sc as plsc

_K = 64
_N = 4096
_GROUPS = 256
_SLABS = _N // _GROUPS


def _f32_to_key(x):
    bits = lax.bitcast_convert_type(x, jnp.int32)
    return jnp.where(bits < 0, bits ^ jnp.int32(0x7FFFFFFF), bits)


def _key_to_f32(k):
    return lax.bitcast_convert_type(
        jnp.where(k < 0, k ^ jnp.int32(0x7FFFFFFF), k), jnp.float32)


def _ceil_mid(lo, hi):
    return (lo >> 1) + (hi >> 1) + ((lo | hi) & 1)


def _prescreen_body(x_ref, o_ref):
    x = x_ref[...]
    r = x.shape[0]
    acc = x[:, 0:_GROUPS]
    for k in range(1, _SLABS):
        acc = jnp.maximum(acc, x[:, k * _GROUPS:(k + 1) * _GROUPS])
    key = _f32_to_key(acc)

    def step(_, lohi):
        lo, hi = lohi
        mid = _ceil_mid(lo, hi)
        cnt = jnp.sum((key >= mid).astype(jnp.float32), axis=1, keepdims=True)
        ok = cnt >= jnp.float32(_K)
        return (jnp.where(ok, mid, lo), jnp.where(ok, hi, mid - jnp.int32(1)))

    lo0 = jnp.full((r, 1), -0x80000000, jnp.int32)
    hi0 = jnp.full((r, 1), 0x7FFFFFFF, jnp.int32)
    lo, _ = jax.lax.fori_loop(0, 16, step, (lo0, hi0))
    o_ref[...] = jnp.broadcast_to(lo, (r, 16))


def _prescreen(x2):
    rows = x2.shape[0]
    blk = 256
    return pl.pallas_call(
        _prescreen_body,
        grid=(rows // blk,),
        in_specs=[pl.BlockSpec((blk, _N), lambda i: (i, 0))],
        out_specs=pl.BlockSpec((blk, 16), lambda i: (i, 0)),
        out_shape=jax.ShapeDtypeStruct((rows, 16), jnp.int32),
    )(x2)


def _sc_topk(x_flat, thr_flat, rows):
    nw = 32
    rows_per = rows // nw
    pairs = rows_per // 2
    mesh = plsc.VectorSubcoreMesh(core_axis_name="c", subcore_axis_name="s")

    @functools.partial(
        pl.kernel,
        mesh=mesh,
        compiler_params=pltpu.CompilerParams(needs_layout_passes=False),
        out_type=jax.ShapeDtypeStruct((rows * _K,), jnp.float32),
        scratch_types=[
            pltpu.VMEM((_N,), jnp.float32),
            pltpu.VMEM((_N,), jnp.float32),
            pltpu.VMEM((_N,), jnp.int32),
            pltpu.VMEM((rows_per * 16,), jnp.int32),
            pltpu.VMEM((64 * _K,), jnp.float32),
            pltpu.VMEM((16,), jnp.int32),
            pltpu.SemaphoreType.DMA,
            pltpu.SemaphoreType.DMA,
        ],
    )
    def sck(x_hbm, thr_hbm, out_hbm, buf0, buf1, cand, thr_v, outstage,
            cnt_v, sem0, sem1):
        wid = lax.axis_index("s") * 2 + lax.axis_index("c")
        row0 = wid * rows_per
        lanes = lax.iota(jnp.int32, 16)
        zero_i = jnp.zeros((16,), jnp.int32)
        one_i = jnp.full((16,), 1, jnp.int32)
        k64 = jnp.full((16,), _K, jnp.int32)
        fifteens = jnp.full((16,), 15, jnp.int32)
        int_min = jnp.full((16,), -0x80000000, jnp.int32)
        int_max = jnp.full((16,), 0x7FFFFFFF, jnp.int32)
        j128 = jnp.full((16,), 128, jnp.int32)

        pltpu.sync_copy(thr_hbm.at[pl.ds(row0 * 16, rows_per * 16)], thr_v)

        def row_src(row):
            return x_hbm.at[pl.ds(row * _N, _N)]

        def splat_last(c):
            return lax.gather(
                c, fifteens[:, None],
                dimension_numbers=lax.GatherDimensionNumbers(
                    offset_dims=(), collapsed_slice_dims=(0,),
                    start_index_map=(0,)),
                slice_sizes=(1,),
                mode=lax.GatherScatterMode.PROMISE_IN_BOUNDS)

        def process(buf, row, rm_vec):
            rloc = row - row0
            lk = thr_v[pl.ds(rloc * 16, 16)]
            for q in range(8):
                cand[pl.ds(q * 16, 16)] = int_min

            def hot(jo, base):
                for u in range(8):
                    x = buf[pl.ds(jo * 128 + u * 16, 16)]
                    uk = _f32_to_key(x)
                    m = uk >= lk
                    c = plsc.cumsum(jnp.where(m, jnp.int32(1), jnp.int32(0)))
                    idx = base + c - 1
                    plsc.store_scatter(cand, [idx], uk, mask=m)
                    base = base + splat_last(c)
                return base

            nsplat = jax.lax.fori_loop(0, _N // 128, hot, zero_i)
            nceil = ((nsplat + 15) >> 4) << 4
            padidx = lanes + nceil - 16
            plsc.store_scatter(cand, [padidx], int_min, mask=padidx >= nsplat)

            kvs = [cand[pl.ds(q * 16, 16)] for q in range(8)]

            def count_ge(mid):
                acc = zero_i
                for q in range(8):
                    acc = acc + jnp.where(kvs[q] >= mid, jnp.int32(1),
                                          jnp.int32(0))

                def cond(st):
                    return jnp.any(st[0] < nceil)

                def body(st):
                    j, a = st
                    kv = plsc.load_gather(cand, [lanes + j])
                    return (j + 16,
                            a + jnp.where(kv >= mid, jnp.int32(1),
                                          jnp.int32(0)))

                _, acc = jax.lax.while_loop(cond, body, (j128, acc))
                return splat_last(plsc.cumsum(acc))

            def bstep(_, lohi):
                lo, hi = lohi
                mid = _ceil_mid(lo, hi)
                ok = count_ge(mid) >= k64
                return (jnp.where(ok, mid, lo),
                        jnp.where(ok, hi, mid - jnp.int32(1)))

            tkey, _ = jax.lax.fori_loop(0, 32, bstep, (int_min, int_max))
            n_ge = count_ge(tkey)

            def emit_fast(kv, ob):
                keep = kv >= tkey
                pc = plsc.cumsum(jnp.where(keep, jnp.int32(1), jnp.int32(0)))
                pos = ob + pc - 1 + rm_vec
                plsc.store_scatter(outstage, [pos], _key_to_f32(kv),
                                   mask=keep)
                return ob + splat_last(pc)

            budget = k64 - count_ge(tkey + 1)

            def emit(kv, eqb, ob):
                gt = kv > tkey
                eq = kv == tkey
                eqc = plsc.cumsum(jnp.where(eq, jnp.int32(1),
                                            jnp.int32(0))) + eqb
                keep = gt | (eq & (eqc <= budget))
                pc = plsc.cumsum(jnp.where(keep, jnp.int32(1), jnp.int32(0)))
                pos = ob + pc - 1 + rm_vec
                plsc.store_scatter(outstage, [pos], _key_to_f32(kv),
                                   mask=keep)
                return splat_last(eqc), ob + splat_last(pc)

            @pl.when(jnp.all(n_ge == k64))
            def _():
                ob = zero_i
                for q in range(8):
                    ob = emit_fast(kvs[q], ob)

                def fcond(st):
                    return jnp.any(st[0] < nceil)

                def fbody(st):
                    j, ob = st
                    kv = plsc.load_gather(cand, [lanes + j])
                    return (j + 16, emit_fast(kv, ob))

                jax.lax.while_loop(fcond, fbody, (j128, ob))

            @pl.when(jnp.any(n_ge != k64))
            def _():
                eqb = zero_i
                ob = zero_i
                for q in range(8):
                    eqb, ob = emit(kvs[q], eqb, ob)

                def fcond(st):
                    return jnp.any(st[0] < nceil)

                def fbody(st):
                    j, eqb, ob = st
                    kv = plsc.load_gather(cand, [lanes + j])
                    eqb, ob = emit(kv, eqb, ob)
                    return (j + 16, eqb, ob)

                jax.lax.while_loop(fcond, fbody, (j128, eqb, ob))

            @pl.when((row & 63) == 63)
            def _():
                pltpu.sync_copy(
                    outstage, out_hbm.at[pl.ds((row - 63) * _K, 64 * _K)])

        pltpu.async_copy(row_src(row0), buf0, sem0)

        def pair(i, rm_vec):
            row_a = row0 + 2 * i
            pltpu.make_async_copy(row_src(row_a), buf0, sem0).wait()
            pltpu.async_copy(row_src(row_a + 1), buf1, sem1)
            process(buf0, row_a, rm_vec)
            pltpu.make_async_copy(row_src(row_a + 1), buf1, sem1).wait()

            @pl.when(i < pairs - 1)
            def _():
                pltpu.async_copy(row_src(row_a + 2), buf0, sem0)

            process(buf1, row_a + 1, rm_vec + _K)
            return (rm_vec + 2 * _K) & jnp.int32(_N - 1)

        jax.lax.fori_loop(0, pairs, pair, zero_i)

    return sck(x_flat, thr_flat)


def kernel(inputs):
    b, s, n = inputs.shape
    rows = b * s
    x2 = inputs.reshape(rows, n)
    thr = _prescreen(x2).reshape(-1)
    out = _sc_topk(x2.reshape(-1), thr, rows)
    return out.reshape(b, s, _K)


# speedup vs baseline: 1.0940x; 1.0151x over previous
import functools

import jax
import jax.numpy as jnp
from jax import lax
from jax.experimental import pallas as pl
from jax.experimental.pallas import tpu as pltpu
from jax.experimental.pallas import tpu_sc as plsc

_K = 64
_N = 4096
_GROUPS = 256
_SLABS = _N // _GROUPS


def _f32_to_key(x):
    bits = lax.bitcast_convert_type(x, jnp.int32)
    return jnp.where(bits < 0, bits ^ jnp.int32(0x7FFFFFFF), bits)


def _key_to_f32(k):
    return lax.bitcast_convert_type(
        jnp.where(k < 0, k ^ jnp.int32(0x7FFFFFFF), k), jnp.float32)


def _ceil_mid(lo, hi):
    return (lo >> 1) + (hi >> 1) + ((lo | hi) & 1)


def _prescreen_body(x_ref, o_ref):
    x = x_ref[...]
    r = x.shape[0]
    acc = x[:, 0:_GROUPS]
    for k in range(1, _SLABS):
        acc = jnp.maximum(acc, x[:, k * _GROUPS:(k + 1) * _GROUPS])
    key = _f32_to_key(acc)

    def step(_, lohi):
        lo, hi = lohi
        mid = _ceil_mid(lo, hi)
        cnt = jnp.sum((key >= mid).astype(jnp.float32), axis=1, keepdims=True)
        ok = cnt >= jnp.float32(_K)
        return (jnp.where(ok, mid, lo), jnp.where(ok, hi, mid - jnp.int32(1)))

    lo0 = jnp.full((r, 1), -0x80000000, jnp.int32)
    hi0 = jnp.full((r, 1), 0x7FFFFFFF, jnp.int32)
    lo, _ = jax.lax.fori_loop(0, 16, step, (lo0, hi0))
    o_ref[...] = jnp.broadcast_to(lo, (r, 16))


def _prescreen(x2):
    rows = x2.shape[0]
    blk = 256
    return pl.pallas_call(
        _prescreen_body,
        grid=(rows // blk,),
        in_specs=[pl.BlockSpec((blk, _N), lambda i: (i, 0))],
        out_specs=pl.BlockSpec((blk, 16), lambda i: (i, 0)),
        out_shape=jax.ShapeDtypeStruct((rows, 16), jnp.int32),
    )(x2)


def _sc_topk(x_flat, thr_flat, rows):
    nw = 32
    rows_per = rows // nw
    pairs = rows_per // 2
    mesh = plsc.VectorSubcoreMesh(core_axis_name="c", subcore_axis_name="s")

    @functools.partial(
        pl.kernel,
        mesh=mesh,
        compiler_params=pltpu.CompilerParams(needs_layout_passes=False),
        out_type=jax.ShapeDtypeStruct((rows * _K,), jnp.float32),
        scratch_types=[
            pltpu.VMEM((_N,), jnp.float32),
            pltpu.VMEM((_N,), jnp.float32),
            pltpu.VMEM((_N,), jnp.int32),
            pltpu.VMEM((rows_per * 16,), jnp.int32),
            pltpu.VMEM((64 * _K,), jnp.float32),
            pltpu.VMEM((16,), jnp.int32),
            pltpu.SemaphoreType.DMA,
            pltpu.SemaphoreType.DMA,
        ],
    )
    def sck(x_hbm, thr_hbm, out_hbm, buf0, buf1, cand, thr_v, outstage,
            cnt_v, sem0, sem1):
        wid = lax.axis_index("s") * 2 + lax.axis_index("c")
        row0 = wid * rows_per
        lanes = lax.iota(jnp.int32, 16)
        zero_i = jnp.zeros((16,), jnp.int32)
        one_i = jnp.full((16,), 1, jnp.int32)
        k64 = jnp.full((16,), _K, jnp.int32)
        fifteens = jnp.full((16,), 15, jnp.int32)
        int_min = jnp.full((16,), -0x80000000, jnp.int32)
        int_max = jnp.full((16,), 0x7FFFFFFF, jnp.int32)
        j128 = jnp.full((16,), 128, jnp.int32)

        pltpu.sync_copy(thr_hbm.at[pl.ds(row0 * 16, rows_per * 16)], thr_v)

        def row_src(row):
            return x_hbm.at[pl.ds(row * _N, _N)]

        def splat_last(c):
            return lax.gather(
                c, fifteens[:, None],
                dimension_numbers=lax.GatherDimensionNumbers(
                    offset_dims=(), collapsed_slice_dims=(0,),
                    start_index_map=(0,)),
                slice_sizes=(1,),
                mode=lax.GatherScatterMode.PROMISE_IN_BOUNDS)

        def process(buf, row, rm_vec):
            rloc = row - row0
            lk = thr_v[pl.ds(rloc * 16, 16)]
            for q in range(8):
                cand[pl.ds(q * 16, 16)] = int_min

            def hot(jo, base):
                for u in range(8):
                    x = buf[pl.ds(jo * 128 + u * 16, 16)]
                    uk = _f32_to_key(x)
                    m = uk >= lk
                    c = plsc.cumsum(jnp.where(m, jnp.int32(1), jnp.int32(0)))
                    idx = base + c - 1
                    plsc.store_scatter(cand, [idx], uk, mask=m)
                    base = base + splat_last(c)
                return base

            nsplat = jax.lax.fori_loop(0, _N // 128, hot, zero_i)
            nceil = ((nsplat + 15) >> 4) << 4
            padidx = lanes + nceil - 16
            plsc.store_scatter(cand, [padidx], int_min, mask=padidx >= nsplat)

            kvs = [cand[pl.ds(q * 16, 16)] for q in range(8)]

            def count_ge(mid):
                acc = zero_i
                for q in range(8):
                    acc = acc + jnp.where(kvs[q] >= mid, jnp.int32(1),
                                          jnp.int32(0))

                def cond(st):
                    return jnp.any(st[0] < nceil)

                def body(st):
                    j, a = st
                    kv = plsc.load_gather(cand, [lanes + j])
                    return (j + 16,
                            a + jnp.where(kv >= mid, jnp.int32(1),
                                          jnp.int32(0)))

                _, acc = jax.lax.while_loop(cond, body, (j128, acc))
                return splat_last(plsc.cumsum(acc))

            def bstep(_, lohi):
                lo, hi = lohi
                mid = _ceil_mid(lo, hi)
                ok = count_ge(mid) >= k64
                return (jnp.where(ok, mid, lo),
                        jnp.where(ok, hi, mid - jnp.int32(1)))

            tkey, _ = jax.lax.fori_loop(0, 32, bstep, (int_min, int_max))
            n_ge = count_ge(tkey)

            def emit_fast(kv, ob):
                keep = kv >= tkey
                pc = plsc.cumsum(jnp.where(keep, jnp.int32(1), jnp.int32(0)))
                pos = ob + pc - 1 + rm_vec
                plsc.store_scatter(outstage, [pos], _key_to_f32(kv),
                                   mask=keep)
                return ob + splat_last(pc)

            @pl.when(jnp.all(n_ge == k64))
            def _():
                ob = zero_i
                for q in range(8):
                    ob = emit_fast(kvs[q], ob)

                def fcond(st):
                    return jnp.any(st[0] < nceil)

                def fbody(st):
                    j, ob = st
                    kv = plsc.load_gather(cand, [lanes + j])
                    return (j + 16, emit_fast(kv, ob))

                jax.lax.while_loop(fcond, fbody, (j128, ob))

            @pl.when(jnp.any(n_ge != k64))
            def _():
                eqb = zero_i
                ob = zero_i
                budget = k64 - count_ge(tkey + 1)

                def emit(kv, eqb, ob):
                    gt = kv > tkey
                    eq = kv == tkey
                    eqc = plsc.cumsum(jnp.where(eq, jnp.int32(1),
                                                jnp.int32(0))) + eqb
                    keep = gt | (eq & (eqc <= budget))
                    pc = plsc.cumsum(jnp.where(keep, jnp.int32(1),
                                               jnp.int32(0)))
                    pos = ob + pc - 1 + rm_vec
                    plsc.store_scatter(outstage, [pos], _key_to_f32(kv),
                                       mask=keep)
                    return splat_last(eqc), ob + splat_last(pc)

                for q in range(8):
                    eqb, ob = emit(kvs[q], eqb, ob)

                def fcond(st):
                    return jnp.any(st[0] < nceil)

                def fbody(st):
                    j, eqb, ob = st
                    kv = plsc.load_gather(cand, [lanes + j])
                    eqb, ob = emit(kv, eqb, ob)
                    return (j + 16, eqb, ob)

                jax.lax.while_loop(fcond, fbody, (j128, eqb, ob))

            @pl.when((row & 63) == 63)
            def _():
                pltpu.sync_copy(
                    outstage, out_hbm.at[pl.ds((row - 63) * _K, 64 * _K)])

        pltpu.async_copy(row_src(row0), buf0, sem0)

        def pair(i, rm_vec):
            row_a = row0 + 2 * i
            pltpu.make_async_copy(row_src(row_a), buf0, sem0).wait()
            pltpu.async_copy(row_src(row_a + 1), buf1, sem1)
            process(buf0, row_a, rm_vec)
            pltpu.make_async_copy(row_src(row_a + 1), buf1, sem1).wait()

            @pl.when(i < pairs - 1)
            def _():
                pltpu.async_copy(row_src(row_a + 2), buf0, sem0)

            process(buf1, row_a + 1, rm_vec + _K)
            return (rm_vec + 2 * _K) & jnp.int32(_N - 1)

        jax.lax.fori_loop(0, pairs, pair, zero_i)

    return sck(x_flat, thr_flat)


def kernel(inputs):
    b, s, n = inputs.shape
    rows = b * s
    x2 = inputs.reshape(rows, n)
    thr = _prescreen(x2).reshape(-1)
    out = _sc_topk(x2.reshape(-1), thr, rows)
    return out.reshape(b, s, _K)


# speedup vs baseline: 1.1012x; 1.0065x over previous
import functools

import jax
import jax.numpy as jnp
from jax import lax
from jax.experimental import pallas as pl
from jax.experimental.pallas import tpu as pltpu
from jax.experimental.pallas import tpu_sc as plsc

_K = 64
_N = 4096
_GROUPS = 256
_SLABS = _N // _GROUPS


def _f32_to_key(x):
    bits = lax.bitcast_convert_type(x, jnp.int32)
    return jnp.where(bits < 0, bits ^ jnp.int32(0x7FFFFFFF), bits)


def _key_to_f32(k):
    return lax.bitcast_convert_type(
        jnp.where(k < 0, k ^ jnp.int32(0x7FFFFFFF), k), jnp.float32)


def _ceil_mid(lo, hi):
    return (lo >> 1) + (hi >> 1) + ((lo | hi) & 1)


def _prescreen_body(x_ref, o_ref):
    x = x_ref[...]
    r = x.shape[0]
    acc = x[:, 0:_GROUPS]
    for k in range(1, _SLABS):
        acc = jnp.maximum(acc, x[:, k * _GROUPS:(k + 1) * _GROUPS])
    key = _f32_to_key(acc)

    def step(_, lohi):
        lo, hi = lohi
        mid = _ceil_mid(lo, hi)
        cnt = jnp.sum((key >= mid).astype(jnp.float32), axis=1, keepdims=True)
        ok = cnt >= jnp.float32(_K)
        return (jnp.where(ok, mid, lo), jnp.where(ok, hi, mid - jnp.int32(1)))

    lo0 = jnp.full((r, 1), -0x80000000, jnp.int32)
    hi0 = jnp.full((r, 1), 0x7FFFFFFF, jnp.int32)
    lo, _ = jax.lax.fori_loop(0, 16, step, (lo0, hi0))
    o_ref[...] = jnp.broadcast_to(lo, (r, 16))


def _prescreen(x2):
    rows = x2.shape[0]
    blk = 256
    return pl.pallas_call(
        _prescreen_body,
        grid=(rows // blk,),
        in_specs=[pl.BlockSpec((blk, _N), lambda i: (i, 0))],
        out_specs=pl.BlockSpec((blk, 16), lambda i: (i, 0)),
        out_shape=jax.ShapeDtypeStruct((rows, 16), jnp.int32),
    )(x2)


def _sc_topk(x_flat, thr_flat, rows):
    nw = 32
    rows_per = rows // nw
    pairs = rows_per // 2
    mesh = plsc.VectorSubcoreMesh(core_axis_name="c", subcore_axis_name="s")

    @functools.partial(
        pl.kernel,
        mesh=mesh,
        compiler_params=pltpu.CompilerParams(needs_layout_passes=False),
        out_type=jax.ShapeDtypeStruct((rows * _K,), jnp.float32),
        scratch_types=[
            pltpu.VMEM((_N,), jnp.float32),
            pltpu.VMEM((_N,), jnp.float32),
            pltpu.VMEM((_N,), jnp.int32),
            pltpu.VMEM((rows_per * 16,), jnp.int32),
            pltpu.VMEM((64 * _K,), jnp.float32),
            pltpu.VMEM((16,), jnp.int32),
            pltpu.SemaphoreType.DMA,
            pltpu.SemaphoreType.DMA,
        ],
    )
    def sck(x_hbm, thr_hbm, out_hbm, buf0, buf1, cand, thr_v, outstage,
            cnt_v, sem0, sem1):
        wid = lax.axis_index("s") * 2 + lax.axis_index("c")
        row0 = wid * rows_per
        lanes = lax.iota(jnp.int32, 16)
        zero_i = jnp.zeros((16,), jnp.int32)
        one_i = jnp.full((16,), 1, jnp.int32)
        k64 = jnp.full((16,), _K, jnp.int32)
        fifteens = jnp.full((16,), 15, jnp.int32)
        int_min = jnp.full((16,), -0x80000000, jnp.int32)
        int_max = jnp.full((16,), 0x7FFFFFFF, jnp.int32)
        j128 = jnp.full((16,), 128, jnp.int32)

        pltpu.sync_copy(thr_hbm.at[pl.ds(row0 * 16, rows_per * 16)], thr_v)

        def row_src(row):
            return x_hbm.at[pl.ds(row * _N, _N)]

        def splat_last(c):
            return lax.gather(
                c, fifteens[:, None],
                dimension_numbers=lax.GatherDimensionNumbers(
                    offset_dims=(), collapsed_slice_dims=(0,),
                    start_index_map=(0,)),
                slice_sizes=(1,),
                mode=lax.GatherScatterMode.PROMISE_IN_BOUNDS)

        def process(buf, row, rm_vec):
            rloc = row - row0
            lk = thr_v[pl.ds(rloc * 16, 16)]
            for q in range(8):
                cand[pl.ds(q * 16, 16)] = int_min

            def hot(jo, base):
                for u in range(8):
                    x = buf[pl.ds(jo * 128 + u * 16, 16)]
                    uk = _f32_to_key(x)
                    m = uk >= lk
                    c = plsc.cumsum(jnp.where(m, jnp.int32(1), jnp.int32(0)))
                    idx = base + c - 1
                    plsc.store_scatter(cand, [idx], uk, mask=m)
                    base = base + splat_last(c)
                return base

            nsplat = jax.lax.fori_loop(0, _N // 128, hot, zero_i)
            nceil = ((nsplat + 15) >> 4) << 4
            padidx = lanes + nceil - 16
            plsc.store_scatter(cand, [padidx], int_min, mask=padidx >= nsplat)

            kvs = [cand[pl.ds(q * 16, 16)] for q in range(8)]

            def count3(m1, m2, m3):
                a1 = a2 = a3 = zero_i
                for q in range(8):
                    a1 = a1 + jnp.where(kvs[q] >= m1, jnp.int32(1),
                                        jnp.int32(0))
                    a2 = a2 + jnp.where(kvs[q] >= m2, jnp.int32(1),
                                        jnp.int32(0))
                    a3 = a3 + jnp.where(kvs[q] >= m3, jnp.int32(1),
                                        jnp.int32(0))

                def cond(st):
                    return jnp.any(st[0] < nceil)

                def body(st):
                    j, b1, b2, b3 = st
                    kv = plsc.load_gather(cand, [lanes + j])
                    return (j + 16,
                            b1 + jnp.where(kv >= m1, jnp.int32(1),
                                           jnp.int32(0)),
                            b2 + jnp.where(kv >= m2, jnp.int32(1),
                                           jnp.int32(0)),
                            b3 + jnp.where(kv >= m3, jnp.int32(1),
                                           jnp.int32(0)))

                _, a1, a2, a3 = jax.lax.while_loop(cond, body,
                                                   (j128, a1, a2, a3))
                return (splat_last(plsc.cumsum(a1)),
                        splat_last(plsc.cumsum(a2)),
                        splat_last(plsc.cumsum(a3)))

            def count_ge(mid):
                c, _, _ = count3(mid, mid, mid)
                return c

            def bstep(_, lohi):
                lo, hi = lohi
                m2 = _ceil_mid(lo, hi)
                m1 = _ceil_mid(lo, m2 - 1)
                m3 = _ceil_mid(m2, hi)
                c1, c2, c3 = count3(m1, m2, m3)
                ok1 = c1 >= k64
                ok2 = c2 >= k64
                ok3 = c3 >= k64
                lo = jnp.where(ok3, m3,
                               jnp.where(ok2, m2, jnp.where(ok1, m1, lo)))
                hi = jnp.where(ok3, hi,
                               jnp.where(ok2, m3 - 1,
                                         jnp.where(ok1, m2 - 1, m1 - 1)))
                return (lo, hi)

            tkey, _ = jax.lax.fori_loop(0, 16, bstep, (int_min, int_max))
            n_ge = count_ge(tkey)

            def emit_fast(kv, ob):
                keep = kv >= tkey
                pc = plsc.cumsum(jnp.where(keep, jnp.int32(1), jnp.int32(0)))
                pos = ob + pc - 1 + rm_vec
                plsc.store_scatter(outstage, [pos], _key_to_f32(kv),
                                   mask=keep)
                return ob + splat_last(pc)

            @pl.when(jnp.all(n_ge == k64))
            def _():
                ob = zero_i
                for q in range(8):
                    ob = emit_fast(kvs[q], ob)

                def fcond(st):
                    return jnp.any(st[0] < nceil)

                def fbody(st):
                    j, ob = st
                    kv = plsc.load_gather(cand, [lanes + j])
                    return (j + 16, emit_fast(kv, ob))

                jax.lax.while_loop(fcond, fbody, (j128, ob))

            @pl.when(jnp.any(n_ge != k64))
            def _():
                eqb = zero_i
                ob = zero_i
                budget = k64 - count_ge(tkey + 1)

                def emit(kv, eqb, ob):
                    gt = kv > tkey
                    eq = kv == tkey
                    eqc = plsc.cumsum(jnp.where(eq, jnp.int32(1),
                                                jnp.int32(0))) + eqb
                    keep = gt | (eq & (eqc <= budget))
                    pc = plsc.cumsum(jnp.where(keep, jnp.int32(1),
                                               jnp.int32(0)))
                    pos = ob + pc - 1 + rm_vec
                    plsc.store_scatter(outstage, [pos], _key_to_f32(kv),
                                       mask=keep)
                    return splat_last(eqc), ob + splat_last(pc)

                for q in range(8):
                    eqb, ob = emit(kvs[q], eqb, ob)

                def fcond(st):
                    return jnp.any(st[0] < nceil)

                def fbody(st):
                    j, eqb, ob = st
                    kv = plsc.load_gather(cand, [lanes + j])
                    eqb, ob = emit(kv, eqb, ob)
                    return (j + 16, eqb, ob)

                jax.lax.while_loop(fcond, fbody, (j128, eqb, ob))

            @pl.when((row & 63) == 63)
            def _():
                pltpu.sync_copy(
                    outstage, out_hbm.at[pl.ds((row - 63) * _K, 64 * _K)])

        pltpu.async_copy(row_src(row0), buf0, sem0)

        def pair(i, rm_vec):
            row_a = row0 + 2 * i
            pltpu.make_async_copy(row_src(row_a), buf0, sem0).wait()
            pltpu.async_copy(row_src(row_a + 1), buf1, sem1)
            process(buf0, row_a, rm_vec)
            pltpu.make_async_copy(row_src(row_a + 1), buf1, sem1).wait()

            @pl.when(i < pairs - 1)
            def _():
                pltpu.async_copy(row_src(row_a + 2), buf0, sem0)

            process(buf1, row_a + 1, rm_vec + _K)
            return (rm_vec + 2 * _K) & jnp.int32(_N - 1)

        jax.lax.fori_loop(0, pairs, pair, zero_i)

    return sck(x_flat, thr_flat)


def kernel(inputs):
    b, s, n = inputs.shape
    rows = b * s
    x2 = inputs.reshape(rows, n)
    thr = _prescreen(x2).reshape(-1)
    out = _sc_topk(x2.reshape(-1), thr, rows)
    return out.reshape(b, s, _K)
